# baseline (speedup 1.0000x reference)
import dataclasses

import numpy as np

import jax
import jax.numpy as jnp
from jax import lax
from jax.experimental import pallas as pl
from jax.experimental.pallas import tpu as pltpu
from jax.experimental.pallas import tpu_sc as plsc

NUM_CORES = 2
NUM_SUBCORES = 16
NUM_TILES = NUM_CORES * NUM_SUBCORES
LANES = 16
GATHER_CHUNK = 128
SLOW_CID = 0
CH_S = 1664
CH_F = 4608
NG_S = CH_S // GATHER_CHUNK
NG_F = CH_F // GATHER_CHUNK
NV_S = CH_S // LANES
NV_F = CH_F // LANES
ROWS0_F = NUM_SUBCORES * CH_S
TOTAL = NUM_SUBCORES * (CH_S + CH_F)
BUF = 128
BUF_VREGS = BUF // LANES
COMPACT_AT = BUF - LANES
KEEP = 30
CAND = NUM_TILES * 32


def _rev(v):
    return lax.rev(v, (0,))


def _sortkv(k, i):
    return plsc.sort_key_val(k, i, descending=True)


def _merge16(ak, ai, bk, bi):
    rbk, rbi = _rev(bk), _rev(bi)
    m = ak >= rbk
    hk = jnp.where(m, ak, rbk)
    hi = jnp.where(m, ai, rbi)
    lk = jnp.where(m, rbk, ak)
    li = jnp.where(m, rbi, ai)
    hk, hi = _sortkv(hk, hi)
    lk, li = _sortkv(lk, li)
    return hk, hi, lk, li


def _merge32(a, b):
    ak0, ai0, ak1, ai1 = a
    bk0, bi0, bk1, bi1 = b
    r0k, r0i = _rev(bk1), _rev(bi1)
    r1k, r1i = _rev(bk0), _rev(bi0)
    m0 = ak0 >= r0k
    c0k = jnp.where(m0, ak0, r0k)
    c0i = jnp.where(m0, ai0, r0i)
    m1 = ak1 >= r1k
    c1k = jnp.where(m1, ak1, r1k)
    c1i = jnp.where(m1, ai1, r1i)
    m2 = c0k >= c1k
    hk = jnp.where(m2, c0k, c1k)
    hi = jnp.where(m2, c0i, c1i)
    lk = jnp.where(m2, c1k, c0k)
    li = jnp.where(m2, c1i, c0i)
    hk, hi = _sortkv(hk, hi)
    lk, li = _sortkv(lk, li)
    return hk, hi, lk, li


def _merge_tree(runs):
    while len(runs) > 1:
        runs = [_merge32(runs[2 * p], runs[2 * p + 1])
                for p in range(len(runs) // 2)]
    return runs[0]


def _compact(bk_ref, bi_ref, bufk_ref, bufi_ref, neg):
    runs16 = []
    for v in range(BUF_VREGS):
        k = bufk_ref[pl.ds(v * LANES, LANES)]
        i = bufi_ref[pl.ds(v * LANES, LANES)]
        runs16.append(_sortkv(k, i))
    runs32 = [_merge16(*runs16[2 * p], *runs16[2 * p + 1])
              for p in range(BUF_VREGS // 2)]
    top = _merge_tree(runs32)
    best = (bk_ref[pl.ds(0, LANES)], bi_ref[pl.ds(0, LANES)],
            bk_ref[pl.ds(LANES, LANES)], bi_ref[pl.ds(LANES, LANES)])
    hk, hi, lk, li = _merge32(best, top)
    bk_ref[pl.ds(0, LANES)] = hk
    bi_ref[pl.ds(0, LANES)] = hi
    bk_ref[pl.ds(LANES, LANES)] = lk
    bi_ref[pl.ds(LANES, LANES)] = li
    for v in range(BUF_VREGS):
        bufk_ref[pl.ds(v * LANES, LANES)] = neg
    return jnp.min(lk)


def _k1_body(n_real, feat_flat_hbm, idx_hbm, candk_hbm, candi_hbm,
             idxs_ref, keys_ref, bufk_ref, bufi_ref, bk_ref, bi_ref, *sems):
    cid = lax.axis_index("c")
    sid = lax.axis_index("s")
    wid = cid * NUM_SUBCORES + sid
    neg = jnp.full((LANES,), -jnp.inf, dtype=jnp.float32)
    zero = jnp.zeros((LANES,), dtype=jnp.int32)
    for v in range(BUF_VREGS):
        bufk_ref[pl.ds(v * LANES, LANES)] = neg
    bk_ref[pl.ds(0, LANES)] = neg
    bk_ref[pl.ds(LANES, LANES)] = neg
    bi_ref[pl.ds(0, LANES)] = zero
    bi_ref[pl.ds(LANES, LANES)] = zero

    lane = lax.iota(jnp.int32, LANES)
    sem = sems[0]

    def side(nchunks, chunk_base, row0, nvr):
        pltpu.sync_copy(
            idx_hbm.at[pl.ds(chunk_base * GATHER_CHUNK,
                             nchunks * GATHER_CHUNK)],
            idxs_ref.at[pl.ds(0, nchunks * GATHER_CHUNK)])
        handles = [
            pltpu.async_copy(
                feat_flat_hbm.at[idxs_ref.at[pl.ds(j * GATHER_CHUNK,
                                                   GATHER_CHUNK)]],
                keys_ref.at[pl.ds(j * GATHER_CHUNK, GATHER_CHUNK)],
                sem)
            for j in range(nchunks)
        ]
        for h in handles:
            h.wait()

        def scan_body(c, carry):
            thr, cnt = carry
            loc = c * LANES + lane
            x = keys_ref[pl.ds(c * LANES, LANES)]
            mask = x > thr
            xi = row0 + loc
            plsc.store_compressed(bufk_ref.at[pl.ds(cnt, LANES)], x,
                                  mask=mask)
            plsc.store_compressed(bufi_ref.at[pl.ds(cnt, LANES)], xi,
                                  mask=mask)
            cnt2 = cnt + plsc.all_reduce_population_count(mask)[0]

            def do_compact():
                new_thr = _compact(bk_ref, bi_ref, bufk_ref, bufi_ref, neg)
                return new_thr, jnp.int32(0)

            return lax.cond(cnt2 > COMPACT_AT, do_compact,
                            lambda: (thr, cnt2))

        lax.fori_loop(0, nvr, scan_body,
                      (jnp.float32(-jnp.inf), jnp.int32(0)))
        _compact(bk_ref, bi_ref, bufk_ref, bufi_ref, neg)
        pltpu.sync_copy(bk_ref, candk_hbm.at[pl.ds(wid * 32, 32)])
        pltpu.sync_copy(bi_ref, candi_hbm.at[pl.ds(wid * 32, 32)])

    last_vr = (n_real - ROWS0_F - (NUM_SUBCORES - 1) * CH_F) // LANES

    @pl.when(cid == SLOW_CID)
    def _():
        side(NG_S, sid * NG_S, sid * CH_S, jnp.int32(NV_S))

    @pl.when(cid != SLOW_CID)
    def _():
        nvr = jnp.where(sid == NUM_SUBCORES - 1, last_vr, NV_F)
        side(NG_F, NUM_SUBCORES * NG_S + sid * NG_F,
             ROWS0_F + sid * CH_F, nvr)


def _k2_body(candk_hbm, candi_hbm, feat_hbm, out_hbm,
             ck_ref, ci_ref, wi_ref, rows_ref, sem):
    cid = lax.axis_index("c")
    sid = lax.axis_index("s")
    wid = cid * NUM_SUBCORES + sid

    @pl.when(wid == 0)
    def _():
        pltpu.sync_copy(candk_hbm, ck_ref)
        pltpu.sync_copy(candi_hbm, ci_ref)
        runs = []
        for t in range(NUM_TILES):
            o = t * 32
            runs.append((ck_ref[pl.ds(o, LANES)], ci_ref[pl.ds(o, LANES)],
                         ck_ref[pl.ds(o + LANES, LANES)],
                         ci_ref[pl.ds(o + LANES, LANES)]))
        hk, hi, lk, li = _merge_tree(runs)
        wi_ref[pl.ds(0, LANES)] = hi
        wi_ref[pl.ds(LANES, LANES)] = li
        pltpu.async_copy(feat_hbm.at[wi_ref], rows_ref, sem).wait()
        pltpu.sync_copy(rows_ref.at[pl.ds(0, KEEP)], out_hbm)


@jax.jit
def kernel(features):
    n, d = features.shape
    mesh = plsc.VectorSubcoreMesh(core_axis_name="c", subcore_axis_name="s")
    cp = pltpu.CompilerParams()
    if "needs_layout_passes" in pltpu.CompilerParams.__dataclass_fields__:
        cp = dataclasses.replace(cp, needs_layout_passes=False)

    rows = np.minimum(np.arange(TOTAL, dtype=np.int32), n - 1)
    idx_all = jnp.asarray(rows * d + (d - 1))
    flat = features.reshape(-1)

    k1 = pl.kernel(
        lambda *args: _k1_body(n, *args),
        out_type=(jax.ShapeDtypeStruct((CAND,), jnp.float32),
                  jax.ShapeDtypeStruct((CAND,), jnp.int32)),
        mesh=mesh,
        scratch_types=[
            pltpu.VMEM((CH_F,), jnp.int32),
            pltpu.VMEM((CH_F,), jnp.float32),
            pltpu.VMEM((BUF,), jnp.float32),
            pltpu.VMEM((BUF,), jnp.int32),
            pltpu.VMEM((32,), jnp.float32),
            pltpu.VMEM((32,), jnp.int32),
            pltpu.SemaphoreType.DMA,
        ],
        compiler_params=cp,
    )
    cand_k, cand_i = k1(flat, idx_all)

    k2 = pl.kernel(
        _k2_body,
        out_type=jax.ShapeDtypeStruct((KEEP, d), jnp.float32),
        mesh=mesh,
        scratch_types=[
            pltpu.VMEM((CAND,), jnp.float32),
            pltpu.VMEM((CAND,), jnp.int32),
            pltpu.VMEM((32,), jnp.int32),
            pltpu.VMEM((32, d), jnp.float32),
            pltpu.SemaphoreType.DMA,
        ],
        compiler_params=cp,
    )
    return k2(cand_k, cand_i, features)


# speedup vs baseline: 1.0347x; 1.0347x over previous
import dataclasses

import numpy as np

import jax
import jax.numpy as jnp
from jax import lax
from jax.experimental import pallas as pl
from jax.experimental.pallas import tpu as pltpu
from jax.experimental.pallas import tpu_sc as plsc

NUM_CORES = 2
NUM_SUBCORES = 16
NUM_TILES = NUM_CORES * NUM_SUBCORES
LANES = 16
GATHER_CHUNK = 128
SLOW_CID = 0
CH_S = 2048
CH_F = 4224
NG_S = CH_S // GATHER_CHUNK
NG_F = CH_F // GATHER_CHUNK
NV_S = CH_S // LANES
NV_F = CH_F // LANES
ROWS0_F = NUM_SUBCORES * CH_S
TOTAL = NUM_SUBCORES * (CH_S + CH_F)
BUF = 128
BUF_VREGS = BUF // LANES
COMPACT_AT = BUF - LANES
KEEP = 30
CAND = NUM_TILES * 32


def _rev(v):
    return lax.rev(v, (0,))


def _sortkv(k, i):
    return plsc.sort_key_val(k, i, descending=True)


def _merge16(ak, ai, bk, bi):
    rbk, rbi = _rev(bk), _rev(bi)
    m = ak >= rbk
    hk = jnp.where(m, ak, rbk)
    hi = jnp.where(m, ai, rbi)
    lk = jnp.where(m, rbk, ak)
    li = jnp.where(m, rbi, ai)
    hk, hi = _sortkv(hk, hi)
    lk, li = _sortkv(lk, li)
    return hk, hi, lk, li


def _merge32(a, b):
    ak0, ai0, ak1, ai1 = a
    bk0, bi0, bk1, bi1 = b
    r0k, r0i = _rev(bk1), _rev(bi1)
    r1k, r1i = _rev(bk0), _rev(bi0)
    m0 = ak0 >= r0k
    c0k = jnp.where(m0, ak0, r0k)
    c0i = jnp.where(m0, ai0, r0i)
    m1 = ak1 >= r1k
    c1k = jnp.where(m1, ak1, r1k)
    c1i = jnp.where(m1, ai1, r1i)
    m2 = c0k >= c1k
    hk = jnp.where(m2, c0k, c1k)
    hi = jnp.where(m2, c0i, c1i)
    lk = jnp.where(m2, c1k, c0k)
    li = jnp.where(m2, c1i, c0i)
    hk, hi = _sortkv(hk, hi)
    lk, li = _sortkv(lk, li)
    return hk, hi, lk, li


def _merge_tree(runs):
    while len(runs) > 1:
        runs = [_merge32(runs[2 * p], runs[2 * p + 1])
                for p in range(len(runs) // 2)]
    return runs[0]


def _compact(bk_ref, bi_ref, bufk_ref, bufi_ref, neg):
    runs16 = []
    for v in range(BUF_VREGS):
        k = bufk_ref[pl.ds(v * LANES, LANES)]
        i = bufi_ref[pl.ds(v * LANES, LANES)]
        runs16.append(_sortkv(k, i))
    runs32 = [_merge16(*runs16[2 * p], *runs16[2 * p + 1])
              for p in range(BUF_VREGS // 2)]
    top = _merge_tree(runs32)
    best = (bk_ref[pl.ds(0, LANES)], bi_ref[pl.ds(0, LANES)],
            bk_ref[pl.ds(LANES, LANES)], bi_ref[pl.ds(LANES, LANES)])
    hk, hi, lk, li = _merge32(best, top)
    bk_ref[pl.ds(0, LANES)] = hk
    bi_ref[pl.ds(0, LANES)] = hi
    bk_ref[pl.ds(LANES, LANES)] = lk
    bi_ref[pl.ds(LANES, LANES)] = li
    for v in range(BUF_VREGS):
        bufk_ref[pl.ds(v * LANES, LANES)] = neg
    return jnp.min(lk)


def _k1_body(n_real, feat_flat_hbm, idx_hbm, candk_hbm, candi_hbm,
             idxs_ref, keys_ref, bufk_ref, bufi_ref, bk_ref, bi_ref, *sems):
    cid = lax.axis_index("c")
    sid = lax.axis_index("s")
    wid = cid * NUM_SUBCORES + sid
    neg = jnp.full((LANES,), -jnp.inf, dtype=jnp.float32)
    zero = jnp.zeros((LANES,), dtype=jnp.int32)
    for v in range(BUF_VREGS):
        bufk_ref[pl.ds(v * LANES, LANES)] = neg
    bk_ref[pl.ds(0, LANES)] = neg
    bk_ref[pl.ds(LANES, LANES)] = neg
    bi_ref[pl.ds(0, LANES)] = zero
    bi_ref[pl.ds(LANES, LANES)] = zero

    lane = lax.iota(jnp.int32, LANES)
    sem = sems[0]

    def side(nchunks, chunk_base, row0, nvr):
        pltpu.sync_copy(
            idx_hbm.at[pl.ds(chunk_base * GATHER_CHUNK,
                             nchunks * GATHER_CHUNK)],
            idxs_ref.at[pl.ds(0, nchunks * GATHER_CHUNK)])
        handles = [
            pltpu.async_copy(
                feat_flat_hbm.at[idxs_ref.at[pl.ds(j * GATHER_CHUNK,
                                                   GATHER_CHUNK)]],
                keys_ref.at[pl.ds(j * GATHER_CHUNK, GATHER_CHUNK)],
                sem)
            for j in range(nchunks)
        ]
        for h in handles:
            h.wait()

        def scan_body(c, carry):
            thr, cnt = carry
            loc = c * LANES + lane
            x = keys_ref[pl.ds(c * LANES, LANES)]
            mask = x > thr
            xi = row0 + loc
            plsc.store_compressed(bufk_ref.at[pl.ds(cnt, LANES)], x,
                                  mask=mask)
            plsc.store_compressed(bufi_ref.at[pl.ds(cnt, LANES)], xi,
                                  mask=mask)
            cnt2 = cnt + plsc.all_reduce_population_count(mask)[0]

            def do_compact():
                new_thr = _compact(bk_ref, bi_ref, bufk_ref, bufi_ref, neg)
                return new_thr, jnp.int32(0)

            return lax.cond(cnt2 > COMPACT_AT, do_compact,
                            lambda: (thr, cnt2))

        lax.fori_loop(0, nvr, scan_body,
                      (jnp.float32(-jnp.inf), jnp.int32(0)))
        _compact(bk_ref, bi_ref, bufk_ref, bufi_ref, neg)
        pltpu.sync_copy(bk_ref, candk_hbm.at[pl.ds(wid * 32, 32)])
        pltpu.sync_copy(bi_ref, candi_hbm.at[pl.ds(wid * 32, 32)])

    last_vr = (n_real - ROWS0_F - (NUM_SUBCORES - 1) * CH_F) // LANES

    @pl.when(cid == SLOW_CID)
    def _():
        side(NG_S, sid * NG_S, sid * CH_S, jnp.int32(NV_S))

    @pl.when(cid != SLOW_CID)
    def _():
        nvr = jnp.where(sid == NUM_SUBCORES - 1, last_vr, NV_F)
        side(NG_F, NUM_SUBCORES * NG_S + sid * NG_F,
             ROWS0_F + sid * CH_F, nvr)


def _k2_body(candk_hbm, candi_hbm, feat_hbm, out_hbm,
             ck_ref, ci_ref, wi_ref, rows_ref, sem):
    cid = lax.axis_index("c")
    sid = lax.axis_index("s")
    wid = cid * NUM_SUBCORES + sid

    @pl.when(wid == 0)
    def _():
        pltpu.sync_copy(candk_hbm, ck_ref)
        pltpu.sync_copy(candi_hbm, ci_ref)
        runs = []
        for t in range(NUM_TILES):
            o = t * 32
            runs.append((ck_ref[pl.ds(o, LANES)], ci_ref[pl.ds(o, LANES)],
                         ck_ref[pl.ds(o + LANES, LANES)],
                         ci_ref[pl.ds(o + LANES, LANES)]))
        hk, hi, lk, li = _merge_tree(runs)
        wi_ref[pl.ds(0, LANES)] = hi
        wi_ref[pl.ds(LANES, LANES)] = li
        pltpu.async_copy(feat_hbm.at[wi_ref], rows_ref, sem).wait()
        pltpu.sync_copy(rows_ref.at[pl.ds(0, KEEP)], out_hbm)


@jax.jit
def kernel(features):
    n, d = features.shape
    mesh = plsc.VectorSubcoreMesh(core_axis_name="c", subcore_axis_name="s")
    cp = pltpu.CompilerParams()
    if "needs_layout_passes" in pltpu.CompilerParams.__dataclass_fields__:
        cp = dataclasses.replace(cp, needs_layout_passes=False)

    rows = np.minimum(np.arange(TOTAL, dtype=np.int32), n - 1)
    idx_all = jnp.asarray(rows * d + (d - 1))
    flat = features.reshape(-1)

    k1 = pl.kernel(
        lambda *args: _k1_body(n, *args),
        out_type=(jax.ShapeDtypeStruct((CAND,), jnp.float32),
                  jax.ShapeDtypeStruct((CAND,), jnp.int32)),
        mesh=mesh,
        scratch_types=[
            pltpu.VMEM((CH_F,), jnp.int32),
            pltpu.VMEM((CH_F,), jnp.float32),
            pltpu.VMEM((BUF,), jnp.float32),
            pltpu.VMEM((BUF,), jnp.int32),
            pltpu.VMEM((32,), jnp.float32),
            pltpu.VMEM((32,), jnp.int32),
            pltpu.SemaphoreType.DMA,
        ],
        compiler_params=cp,
    )
    cand_k, cand_i = k1(flat, idx_all)

    k2 = pl.kernel(
        _k2_body,
        out_type=jax.ShapeDtypeStruct((KEEP, d), jnp.float32),
        mesh=mesh,
        scratch_types=[
            pltpu.VMEM((CAND,), jnp.float32),
            pltpu.VMEM((CAND,), jnp.int32),
            pltpu.VMEM((32,), jnp.int32),
            pltpu.VMEM((32, d), jnp.float32),
            pltpu.SemaphoreType.DMA,
        ],
        compiler_params=cp,
    )
    return k2(cand_k, cand_i, features)


# speedup vs baseline: 1.0437x; 1.0087x over previous
import dataclasses

import numpy as np

import jax
import jax.numpy as jnp
from jax import lax
from jax.experimental import pallas as pl
from jax.experimental.pallas import tpu as pltpu
from jax.experimental.pallas import tpu_sc as plsc

NUM_CORES = 2
NUM_SUBCORES = 16
NUM_TILES = NUM_CORES * NUM_SUBCORES
LANES = 16
GATHER_CHUNK = 128
SLOW_CID = 0
CH_S = 2304
CH_F = 3968
NG_S = CH_S // GATHER_CHUNK
NG_F = CH_F // GATHER_CHUNK
NV_S = CH_S // LANES
NV_F = CH_F // LANES
ROWS0_F = NUM_SUBCORES * CH_S
TOTAL = NUM_SUBCORES * (CH_S + CH_F)
BUF = 128
BUF_VREGS = BUF // LANES
COMPACT_AT = BUF - LANES
KEEP = 30
CAND = NUM_TILES * 32


def _rev(v):
    return lax.rev(v, (0,))


def _sortkv(k, i):
    return plsc.sort_key_val(k, i, descending=True)


def _merge16(ak, ai, bk, bi):
    rbk, rbi = _rev(bk), _rev(bi)
    m = ak >= rbk
    hk = jnp.where(m, ak, rbk)
    hi = jnp.where(m, ai, rbi)
    lk = jnp.where(m, rbk, ak)
    li = jnp.where(m, rbi, ai)
    hk, hi = _sortkv(hk, hi)
    lk, li = _sortkv(lk, li)
    return hk, hi, lk, li


def _merge32(a, b):
    ak0, ai0, ak1, ai1 = a
    bk0, bi0, bk1, bi1 = b
    r0k, r0i = _rev(bk1), _rev(bi1)
    r1k, r1i = _rev(bk0), _rev(bi0)
    m0 = ak0 >= r0k
    c0k = jnp.where(m0, ak0, r0k)
    c0i = jnp.where(m0, ai0, r0i)
    m1 = ak1 >= r1k
    c1k = jnp.where(m1, ak1, r1k)
    c1i = jnp.where(m1, ai1, r1i)
    m2 = c0k >= c1k
    hk = jnp.where(m2, c0k, c1k)
    hi = jnp.where(m2, c0i, c1i)
    lk = jnp.where(m2, c1k, c0k)
    li = jnp.where(m2, c1i, c0i)
    hk, hi = _sortkv(hk, hi)
    lk, li = _sortkv(lk, li)
    return hk, hi, lk, li


def _merge_tree(runs):
    while len(runs) > 1:
        runs = [_merge32(runs[2 * p], runs[2 * p + 1])
                for p in range(len(runs) // 2)]
    return runs[0]


def _compact(bk_ref, bi_ref, bufk_ref, bufi_ref, neg):
    runs16 = []
    for v in range(BUF_VREGS):
        k = bufk_ref[pl.ds(v * LANES, LANES)]
        i = bufi_ref[pl.ds(v * LANES, LANES)]
        runs16.append(_sortkv(k, i))
    runs32 = [_merge16(*runs16[2 * p], *runs16[2 * p + 1])
              for p in range(BUF_VREGS // 2)]
    top = _merge_tree(runs32)
    best = (bk_ref[pl.ds(0, LANES)], bi_ref[pl.ds(0, LANES)],
            bk_ref[pl.ds(LANES, LANES)], bi_ref[pl.ds(LANES, LANES)])
    hk, hi, lk, li = _merge32(best, top)
    bk_ref[pl.ds(0, LANES)] = hk
    bi_ref[pl.ds(0, LANES)] = hi
    bk_ref[pl.ds(LANES, LANES)] = lk
    bi_ref[pl.ds(LANES, LANES)] = li
    for v in range(BUF_VREGS):
        bufk_ref[pl.ds(v * LANES, LANES)] = neg
    return jnp.min(lk)


def _k1_body(n_real, feat_flat_hbm, idx_hbm, candk_hbm, candi_hbm,
             idxs_ref, keys_ref, bufk_ref, bufi_ref, bk_ref, bi_ref, *sems):
    cid = lax.axis_index("c")
    sid = lax.axis_index("s")
    wid = cid * NUM_SUBCORES + sid
    neg = jnp.full((LANES,), -jnp.inf, dtype=jnp.float32)
    zero = jnp.zeros((LANES,), dtype=jnp.int32)
    for v in range(BUF_VREGS):
        bufk_ref[pl.ds(v * LANES, LANES)] = neg
    bk_ref[pl.ds(0, LANES)] = neg
    bk_ref[pl.ds(LANES, LANES)] = neg
    bi_ref[pl.ds(0, LANES)] = zero
    bi_ref[pl.ds(LANES, LANES)] = zero

    lane = lax.iota(jnp.int32, LANES)
    sem = sems[0]

    def side(nchunks, chunk_base, row0, nvr):
        pltpu.sync_copy(
            idx_hbm.at[pl.ds(chunk_base * GATHER_CHUNK,
                             nchunks * GATHER_CHUNK)],
            idxs_ref.at[pl.ds(0, nchunks * GATHER_CHUNK)])
        handles = [
            pltpu.async_copy(
                feat_flat_hbm.at[idxs_ref.at[pl.ds(j * GATHER_CHUNK,
                                                   GATHER_CHUNK)]],
                keys_ref.at[pl.ds(j * GATHER_CHUNK, GATHER_CHUNK)],
                sem)
            for j in range(nchunks)
        ]
        for h in handles:
            h.wait()

        def scan_body(c, carry):
            thr, cnt = carry
            loc = c * LANES + lane
            x = keys_ref[pl.ds(c * LANES, LANES)]
            mask = x > thr
            xi = row0 + loc
            plsc.store_compressed(bufk_ref.at[pl.ds(cnt, LANES)], x,
                                  mask=mask)
            plsc.store_compressed(bufi_ref.at[pl.ds(cnt, LANES)], xi,
                                  mask=mask)
            cnt2 = cnt + plsc.all_reduce_population_count(mask)[0]

            def do_compact():
                new_thr = _compact(bk_ref, bi_ref, bufk_ref, bufi_ref, neg)
                return new_thr, jnp.int32(0)

            return lax.cond(cnt2 > COMPACT_AT, do_compact,
                            lambda: (thr, cnt2))

        lax.fori_loop(0, nvr, scan_body,
                      (jnp.float32(-jnp.inf), jnp.int32(0)))
        _compact(bk_ref, bi_ref, bufk_ref, bufi_ref, neg)
        pltpu.sync_copy(bk_ref, candk_hbm.at[pl.ds(wid * 32, 32)])
        pltpu.sync_copy(bi_ref, candi_hbm.at[pl.ds(wid * 32, 32)])

    last_vr = (n_real - ROWS0_F - (NUM_SUBCORES - 1) * CH_F) // LANES

    @pl.when(cid == SLOW_CID)
    def _():
        side(NG_S, sid * NG_S, sid * CH_S, jnp.int32(NV_S))

    @pl.when(cid != SLOW_CID)
    def _():
        nvr = jnp.where(sid == NUM_SUBCORES - 1, last_vr, NV_F)
        side(NG_F, NUM_SUBCORES * NG_S + sid * NG_F,
             ROWS0_F + sid * CH_F, nvr)


def _k2_body(candk_hbm, candi_hbm, feat_hbm, out_hbm,
             ck_ref, ci_ref, wi_ref, rows_ref, sem):
    cid = lax.axis_index("c")
    sid = lax.axis_index("s")
    wid = cid * NUM_SUBCORES + sid

    @pl.when(wid == 0)
    def _():
        pltpu.sync_copy(candk_hbm, ck_ref)
        pltpu.sync_copy(candi_hbm, ci_ref)
        runs = []
        for t in range(NUM_TILES):
            o = t * 32
            runs.append((ck_ref[pl.ds(o, LANES)], ci_ref[pl.ds(o, LANES)],
                         ck_ref[pl.ds(o + LANES, LANES)],
                         ci_ref[pl.ds(o + LANES, LANES)]))
        hk, hi, lk, li = _merge_tree(runs)
        wi_ref[pl.ds(0, LANES)] = hi
        wi_ref[pl.ds(LANES, LANES)] = li
        pltpu.async_copy(feat_hbm.at[wi_ref], rows_ref, sem).wait()
        pltpu.sync_copy(rows_ref.at[pl.ds(0, KEEP)], out_hbm)


@jax.jit
def kernel(features):
    n, d = features.shape
    mesh = plsc.VectorSubcoreMesh(core_axis_name="c", subcore_axis_name="s")
    cp = pltpu.CompilerParams()
    if "needs_layout_passes" in pltpu.CompilerParams.__dataclass_fields__:
        cp = dataclasses.replace(cp, needs_layout_passes=False)

    rows = np.minimum(np.arange(TOTAL, dtype=np.int32), n - 1)
    idx_all = jnp.asarray(rows * d + (d - 1))
    flat = features.reshape(-1)

    k1 = pl.kernel(
        lambda *args: _k1_body(n, *args),
        out_type=(jax.ShapeDtypeStruct((CAND,), jnp.float32),
                  jax.ShapeDtypeStruct((CAND,), jnp.int32)),
        mesh=mesh,
        scratch_types=[
            pltpu.VMEM((CH_F,), jnp.int32),
            pltpu.VMEM((CH_F,), jnp.float32),
            pltpu.VMEM((BUF,), jnp.float32),
            pltpu.VMEM((BUF,), jnp.int32),
            pltpu.VMEM((32,), jnp.float32),
            pltpu.VMEM((32,), jnp.int32),
            pltpu.SemaphoreType.DMA,
        ],
        compiler_params=cp,
    )
    cand_k, cand_i = k1(flat, idx_all)

    k2 = pl.kernel(
        _k2_body,
        out_type=jax.ShapeDtypeStruct((KEEP, d), jnp.float32),
        mesh=mesh,
        scratch_types=[
            pltpu.VMEM((CAND,), jnp.float32),
            pltpu.VMEM((CAND,), jnp.int32),
            pltpu.VMEM((32,), jnp.int32),
            pltpu.VMEM((32, d), jnp.float32),
            pltpu.SemaphoreType.DMA,
        ],
        compiler_params=cp,
    )
    return k2(cand_k, cand_i, features)
